# baseline (speedup 1.0000x reference)
import functools

import jax
import jax.numpy as jnp
from jax import lax
from jax.experimental import pallas as pl
from jax.experimental.pallas import tpu as pltpu
from jax.experimental.pallas import tpu_sc as plsc

_HIDDEN = 1024
_E = 16
_FFN = 4096
_NTOK = 256
_FCHUNK = 2048
_NF = _FFN // _FCHUNK

_NC = 2
_NS = 16
_TPW = _NTOK // (_NC * _NS)


def _logits_body(x_ref, wr_ref, out_ref):
    out_ref[...] = jax.lax.dot_general(
        x_ref[...], wr_ref[...], (((1,), (1,)), ((), ())),
        preferred_element_type=jnp.float32,
    )


def _router_logits(x2d, Wr):
    return pl.pallas_call(
        _logits_body,
        out_shape=jax.ShapeDtypeStruct((_NTOK, _E), jnp.float32),
    )(x2d, Wr)


def _sc_route_body(logits_hbm, wts_hbm, lg_v, w_v):
    wid = lax.axis_index("s") * _NC + lax.axis_index("c")
    base = wid * _TPW
    pltpu.sync_copy(logits_hbm.at[pl.ds(base, _TPW), :], lg_v)
    iota = lax.iota(jnp.int32, _E)

    def perm(v, idx):
        return lax.gather(
            v, idx[:, None],
            dimension_numbers=lax.GatherDimensionNumbers(
                offset_dims=(), collapsed_slice_dims=(0,),
                start_index_map=(0,)),
            slice_sizes=(1,),
            mode=lax.GatherScatterMode.PROMISE_IN_BOUNDS,
        )

    def allmax(v):
        for sh in (8, 4, 2, 1):
            v = jnp.maximum(v, perm(v, iota ^ sh))
        return v

    def allmin(v):
        for sh in (8, 4, 2, 1):
            v = jnp.minimum(v, perm(v, iota ^ sh))
        return v

    for i in range(_TPW):
        row = lg_v[i]
        m1 = allmax(row)
        i1 = allmin(jnp.where(row == m1, iota, _E))
        sel1 = iota == i1
        masked = jnp.where(sel1, -jnp.inf, row)
        m2 = allmax(masked)
        i2 = allmin(jnp.where(masked == m2, iota, _E))
        sel2 = iota == i2
        t = jnp.exp(m2 - m1)
        p1 = 1.0 / (1.0 + t)
        p2 = t / (1.0 + t)
        w_v[i] = jnp.where(sel1, p1, 0.0) + jnp.where(sel2, p2, 0.0)
    pltpu.sync_copy(w_v, wts_hbm.at[pl.ds(base, _TPW), :])


@functools.partial(
    pl.kernel,
    out_type=jax.ShapeDtypeStruct((_NTOK, _E), jnp.float32),
    mesh=plsc.VectorSubcoreMesh(core_axis_name="c", subcore_axis_name="s"),
    scratch_types=[
        pltpu.VMEM((_TPW, _E), jnp.float32),
        pltpu.VMEM((_TPW, _E), jnp.float32),
    ],
)
def _sc_route(logits_hbm, wts_hbm, lg_v, w_v):
    _sc_route_body(logits_hbm, wts_hbm, lg_v, w_v)


def _ffn_body(x_ref, wts_ref, w1_ref, b1_ref, w2_ref, b2_ref, out_ref):
    e = pl.program_id(0)
    f = pl.program_id(1)
    lane = jax.lax.broadcasted_iota(jnp.int32, (_NTOK, _E), 1)

    xb = x_ref[...].astype(jnp.bfloat16)
    h = jax.lax.dot_general(
        xb, w1_ref[0].astype(jnp.bfloat16), (((1,), (1,)), ((), ())),
        preferred_element_type=jnp.float32,
    )
    h = h + b1_ref[0]
    a = 0.5 * h * (1.0 + jax.lax.erf(h * 0.7071067811865476))
    o = jax.lax.dot_general(
        a.astype(jnp.bfloat16), w2_ref[0].astype(jnp.bfloat16),
        (((1,), (1,)), ((), ())),
        preferred_element_type=jnp.float32,
    )
    o = jnp.where(f == 0, o + b2_ref[0], o)
    wcol = jnp.sum(wts_ref[...] * (lane == e).astype(jnp.float32),
                   axis=1, keepdims=True)
    contrib = wcol * o

    @pl.when((e == 0) & (f == 0))
    def _init():
        out_ref[...] = contrib

    @pl.when(~((e == 0) & (f == 0)))
    def _acc():
        out_ref[...] += contrib


def kernel(x, Wr, W1, b1, W2, b2):
    B, S, D = x.shape
    xf = x.reshape(B * S, D)
    logits = _router_logits(xf, Wr)
    wts = _sc_route(logits)
    b1r = b1.reshape(_E * _NF, 1, _FCHUNK)
    b2r = b2.reshape(_E, 1, _HIDDEN)
    out = pl.pallas_call(
        _ffn_body,
        grid=(_E, _NF),
        in_specs=[
            pl.BlockSpec((_NTOK, _HIDDEN), lambda e, f: (0, 0)),
            pl.BlockSpec((_NTOK, _E), lambda e, f: (0, 0)),
            pl.BlockSpec((1, _FCHUNK, _HIDDEN), lambda e, f: (e, f, 0)),
            pl.BlockSpec((1, 1, _FCHUNK), lambda e, f: (e * _NF + f, 0, 0)),
            pl.BlockSpec((1, _HIDDEN, _FCHUNK), lambda e, f: (e, 0, f)),
            pl.BlockSpec((1, 1, _HIDDEN), lambda e, f: (e, 0, 0)),
        ],
        out_specs=pl.BlockSpec((_NTOK, _HIDDEN), lambda e, f: (0, 0)),
        out_shape=jax.ShapeDtypeStruct((_NTOK, _HIDDEN), jnp.float32),
        compiler_params=pltpu.CompilerParams(
            dimension_semantics=("arbitrary", "arbitrary"),
        ),
    )(xf, wts, W1, b1r, W2, b2r)
    return out.reshape(B, S, D)
